# baseline (speedup 1.0000x reference)
import jax
import jax.numpy as jnp
from jax.experimental import pallas as pl

_ROWS = 2000


def _split_kernel(x_ref, o0_ref, o1_ref):
    x = x_ref[...]
    o0_ref[...] = jnp.concatenate(
        [x[:, 0:64], x[:, 128:224], x[:, 320:400]], axis=1)
    o1_ref[...] = jnp.concatenate(
        [x[:, 64:128], x[:, 224:320], x[:, 400:480]], axis=1)


def kernel(x):
    n, c = x.shape
    grid = (n // _ROWS,)
    out_sd = jax.ShapeDtypeStruct((n, 240), x.dtype)
    o0, o1 = pl.pallas_call(
        _split_kernel,
        grid=grid,
        in_specs=[pl.BlockSpec((_ROWS, c), lambda i: (i, 0))],
        out_specs=[
            pl.BlockSpec((_ROWS, 240), lambda i: (i, 0)),
            pl.BlockSpec((_ROWS, 240), lambda i: (i, 0)),
        ],
        out_shape=[out_sd, out_sd],
    )(x)
    return (o0, o1)
